# Initial kernel scaffold; baseline (speedup 1.0000x reference)
import jax
import jax.numpy as jnp
from jax.experimental import pallas as pl


def kernel(x, pos, batch, W1, b1, g1, be1, W2, b2, g2, be2, Wl, bl, g3, be3, wc1, bc1, g4, be4, wc2, bc2, g5, be5, wdw, bdw, Wlin, blin, Wout, bout):
    raise NotImplementedError("write your pallas kernel here")


# speedup vs baseline: 2.6287x; 2.6287x over previous
import functools

import jax
import jax.numpy as jnp
from jax.experimental import pallas as pl
from jax.experimental.pallas import tpu as pltpu
from jax.experimental.pallas import tpu_sc as plsc

N = 10000
D = 3
K = 16
C_IN = 128
C_DEL = 64
C_TOT = C_IN + C_DEL
C_OUT = 128
N_CLS = 40
N_GRAPHS = 16
EPS = 1e-5
NPAD = 10240

R_KNN = 200
NB = 400
RB = NB * K
G_KNN = N // R_KNN
G_P = N // NB

CH = 128
NW = 32
NCHUNK = (N * K) // CH
SC_ITERS = (NCHUNK + NW - 1) // NW


def _elu(x):
    return jnp.where(x > 0, x, jnp.exp(x) - 1.0)


def _knn_body(prow, pcolT, brow, bcol, idx_out):
    p = prow[...]
    pt = pcolT[...]
    sqr = jnp.sum(p * p, axis=1, keepdims=True)
    sqc = jnp.sum(pt * pt, axis=0, keepdims=True)
    dot = jnp.dot(p, pt, preferred_element_type=jnp.float32)
    d2 = sqr + sqc - 2.0 * dot
    big = jnp.float32(jnp.inf)
    d2 = jnp.where(brow[...] == bcol[...], d2, big)
    iota = jax.lax.broadcasted_iota(jnp.int32, (R_KNN, NPAD), 1)
    taken = jnp.zeros((R_KNN, NPAD), jnp.bool_)
    cols = []
    for _ in range(K):
        m = jnp.min(d2, axis=1, keepdims=True)
        sel = jnp.where((d2 == m) & (~taken), iota, jnp.int32(NPAD))
        ig = jnp.min(sel, axis=1, keepdims=True)
        cols.append(ig)
        hit = iota == ig
        taken = taken | hit
        d2 = jnp.where(hit, big, d2)
    idx_out[...] = jnp.concatenate(cols, axis=1)


def _knn(pos16, posT_pad, brow, bcol):
    return pl.pallas_call(
        _knn_body,
        grid=(G_KNN,),
        in_specs=[
            pl.BlockSpec((R_KNN, 16), lambda i: (i, 0)),
            pl.BlockSpec((16, NPAD), lambda i: (0, 0)),
            pl.BlockSpec((R_KNN, 1), lambda i: (i, 0)),
            pl.BlockSpec((1, NPAD), lambda i: (0, 0)),
        ],
        out_specs=pl.BlockSpec((R_KNN, K), lambda i: (i, 0)),
        out_shape=jax.ShapeDtypeStruct((N, K), jnp.int32),
    )(pos16, posT_pad, brow, bcol)


def _make_sc_gather(feat):
    mesh = plsc.VectorSubcoreMesh(core_axis_name="c", subcore_axis_name="s")

    @functools.partial(
        pl.kernel,
        out_type=jax.ShapeDtypeStruct((N * K, feat), jnp.float32),
        mesh=mesh,
        scratch_types=[
            pltpu.VMEM((CH,), jnp.int32),
            pltpu.VMEM((CH, feat), jnp.float32),
            pltpu.SemaphoreType.DMA,
        ],
    )
    def gather(table_hbm, col_hbm, out_hbm, idx_v, rows_v, sem):
        wid = jax.lax.axis_index("s") * 2 + jax.lax.axis_index("c")

        def body(j, carry):
            c = j * NW + wid

            @pl.when(c < NCHUNK)
            def _():
                base = c * CH
                pltpu.sync_copy(col_hbm.at[pl.ds(base, CH)], idx_v)
                pltpu.async_copy(table_hbm.at[idx_v], rows_v, sem).wait()
                pltpu.sync_copy(rows_v, out_hbm.at[pl.ds(base, CH)])

            return carry

        jax.lax.fori_loop(0, SC_ITERS, body, 0)

    return gather


def _sc_gather_pos(table, col):
    return _make_sc_gather(C_IN)(table, col)


def _sc_gather_x(table, col):
    return _make_sc_gather(C_IN)(table, col)


def _p0_body(pg128, pg_o):
    pg_o[...] = pg128[...][:, :16]


def _p0(pg128):
    return pl.pallas_call(
        _p0_body,
        grid=(G_P,),
        in_specs=[pl.BlockSpec((RB, C_IN), lambda i: (i, 0))],
        out_specs=pl.BlockSpec((RB, 16), lambda i: (i, 0)),
        out_shape=jax.ShapeDtypeStruct((N * K, 16), jnp.float32),
    )(pg128)


def _p1_body(pg16, pg256, pr16, pr256, W1r, b1r, Wlr, blr,
             a1_o, u_o, s1_o, q1_o, su_o, qu_o):
    pd16 = pg16[...] - pr16[...]
    a1 = _elu(jnp.dot(pd16, W1r[...], preferred_element_type=jnp.float32)
              + b1r[...])
    a1_o[...] = a1
    pd256 = pg256[...] - pr256[...]
    u = _elu(jnp.dot(pd256, Wlr[...], preferred_element_type=jnp.float32)
             + blr[...])
    u_o[...] = u

    @pl.when(pl.program_id(0) == 0)
    def _():
        s1_o[...] = jnp.zeros_like(s1_o)
        q1_o[...] = jnp.zeros_like(q1_o)
        su_o[...] = jnp.zeros_like(su_o)
        qu_o[...] = jnp.zeros_like(qu_o)

    s1_o[...] += jnp.sum(a1, axis=0, keepdims=True)
    q1_o[...] += jnp.sum(a1 * a1, axis=0, keepdims=True)
    su_o[...] += jnp.sum(u, axis=0, keepdims=True)
    qu_o[...] += jnp.sum(u * u, axis=0, keepdims=True)


def _p1(pg, pr16, pr256, W1_16, b1r, Wl_exp, blr):
    return pl.pallas_call(
        _p1_body,
        grid=(G_P,),
        in_specs=[
            pl.BlockSpec((RB, 16), lambda i: (i, 0)),
            pl.BlockSpec((NB, 256), lambda i: (i, 0)),
            pl.BlockSpec((RB, 16), lambda i: (i, 0)),
            pl.BlockSpec((NB, 256), lambda i: (i, 0)),
            pl.BlockSpec((16, C_DEL), lambda i: (0, 0)),
            pl.BlockSpec((1, C_DEL), lambda i: (0, 0)),
            pl.BlockSpec((256, 256), lambda i: (0, 0)),
            pl.BlockSpec((1, 256), lambda i: (0, 0)),
        ],
        out_specs=[
            pl.BlockSpec((RB, C_DEL), lambda i: (i, 0)),
            pl.BlockSpec((NB, 256), lambda i: (i, 0)),
            pl.BlockSpec((1, C_DEL), lambda i: (0, 0)),
            pl.BlockSpec((1, C_DEL), lambda i: (0, 0)),
            pl.BlockSpec((1, 256), lambda i: (0, 0)),
            pl.BlockSpec((1, 256), lambda i: (0, 0)),
        ],
        out_shape=[
            jax.ShapeDtypeStruct((N * K, C_DEL), jnp.float32),
            jax.ShapeDtypeStruct((N, 256), jnp.float32),
            jax.ShapeDtypeStruct((1, C_DEL), jnp.float32),
            jax.ShapeDtypeStruct((1, C_DEL), jnp.float32),
            jax.ShapeDtypeStruct((1, 256), jnp.float32),
            jax.ShapeDtypeStruct((1, 256), jnp.float32),
        ],
    )(pg, pg.reshape(N, 256), pr16, pr256, W1_16, b1r, Wl_exp, blr)


def _p2_body(a1r, ur, sc1, sh1, W2r, b2r, sc3, sh3, Wb1, bc1r,
             a2_o, v_o, s2_o, q2_o, sv_o, qv_o):
    h1 = a1r[...] * sc1[...] + sh1[...]
    a2 = _elu(jnp.dot(h1, W2r[...], preferred_element_type=jnp.float32)
              + b2r[...])
    a2_o[...] = a2
    ub = ur[...] * sc3[...] + sh3[...]
    v = _elu(jnp.dot(ub, Wb1[...], preferred_element_type=jnp.float32)
             + bc1r[...])
    v_o[...] = v

    @pl.when(pl.program_id(0) == 0)
    def _():
        s2_o[...] = jnp.zeros_like(s2_o)
        q2_o[...] = jnp.zeros_like(q2_o)
        sv_o[...] = jnp.zeros_like(sv_o)
        qv_o[...] = jnp.zeros_like(qv_o)

    s2_o[...] += jnp.sum(a2, axis=0, keepdims=True)
    q2_o[...] += jnp.sum(a2 * a2, axis=0, keepdims=True)
    sv_o[...] += jnp.sum(v, axis=0, keepdims=True)
    qv_o[...] += jnp.sum(v * v, axis=0, keepdims=True)


def _p2(a1, u, sc1, sh1, W2, b2r, sc3, sh3, Wbig1, bc1r):
    return pl.pallas_call(
        _p2_body,
        grid=(G_P,),
        in_specs=[
            pl.BlockSpec((RB, C_DEL), lambda i: (i, 0)),
            pl.BlockSpec((NB, 256), lambda i: (i, 0)),
            pl.BlockSpec((1, C_DEL), lambda i: (0, 0)),
            pl.BlockSpec((1, C_DEL), lambda i: (0, 0)),
            pl.BlockSpec((C_DEL, C_DEL), lambda i: (0, 0)),
            pl.BlockSpec((1, C_DEL), lambda i: (0, 0)),
            pl.BlockSpec((1, 256), lambda i: (0, 0)),
            pl.BlockSpec((1, 256), lambda i: (0, 0)),
            pl.BlockSpec((256, 256), lambda i: (0, 0)),
            pl.BlockSpec((1, 256), lambda i: (0, 0)),
        ],
        out_specs=[
            pl.BlockSpec((RB, C_DEL), lambda i: (i, 0)),
            pl.BlockSpec((NB, 256), lambda i: (i, 0)),
            pl.BlockSpec((1, C_DEL), lambda i: (0, 0)),
            pl.BlockSpec((1, C_DEL), lambda i: (0, 0)),
            pl.BlockSpec((1, 256), lambda i: (0, 0)),
            pl.BlockSpec((1, 256), lambda i: (0, 0)),
        ],
        out_shape=[
            jax.ShapeDtypeStruct((N * K, C_DEL), jnp.float32),
            jax.ShapeDtypeStruct((N, 256), jnp.float32),
            jax.ShapeDtypeStruct((1, C_DEL), jnp.float32),
            jax.ShapeDtypeStruct((1, C_DEL), jnp.float32),
            jax.ShapeDtypeStruct((1, 256), jnp.float32),
            jax.ShapeDtypeStruct((1, 256), jnp.float32),
        ],
    )(a1, u, sc1, sh1, W2, b2r, sc3, sh3, Wbig1, bc1r)


def _p3_body(vr, sc4, sh4, Wb2, bc2r, w_o, sw_o, qw_o):
    vb = vr[...] * sc4[...] + sh4[...]
    w = jnp.dot(vb, Wb2[...], preferred_element_type=jnp.float32) + bc2r[...]
    w_o[...] = w

    @pl.when(pl.program_id(0) == 0)
    def _():
        sw_o[...] = jnp.zeros_like(sw_o)
        qw_o[...] = jnp.zeros_like(qw_o)

    sw_o[...] += jnp.sum(w, axis=0, keepdims=True)
    qw_o[...] += jnp.sum(w * w, axis=0, keepdims=True)


def _p3(v, sc4, sh4, Wbig2, bc2r):
    return pl.pallas_call(
        _p3_body,
        grid=(G_P,),
        in_specs=[
            pl.BlockSpec((NB, 256), lambda i: (i, 0)),
            pl.BlockSpec((1, 256), lambda i: (0, 0)),
            pl.BlockSpec((1, 256), lambda i: (0, 0)),
            pl.BlockSpec((256, 256), lambda i: (0, 0)),
            pl.BlockSpec((1, 256), lambda i: (0, 0)),
        ],
        out_specs=[
            pl.BlockSpec((NB, 256), lambda i: (i, 0)),
            pl.BlockSpec((1, 256), lambda i: (0, 0)),
            pl.BlockSpec((1, 256), lambda i: (0, 0)),
        ],
        out_shape=[
            jax.ShapeDtypeStruct((N, 256), jnp.float32),
            jax.ShapeDtypeStruct((1, 256), jnp.float32),
            jax.ShapeDtypeStruct((1, 256), jnp.float32),
        ],
    )(v, sc4, sh4, Wbig2, bc2r)


def _p4_body(a2v, wr, xgv, br, sc2, sh2, sc5, sh5, wdwT,
             bdw_a, bdw_b, Wlin_a, Wlin_b, blinr, Woutr, boutr,
             out_o, psum, cnt):
    Tm = wr[...] * sc5[...] + sh5[...]
    a2b = a2v[...]
    xgb = xgv[...]
    wdwTb = wdwT[...]
    z64 = jnp.zeros((NB, C_DEL), jnp.float32)
    z128 = jnp.zeros((NB, C_IN), jnp.float32)
    for g in range(K):
        Sg = jnp.dot(Tm[:, g * 16:(g + 1) * 16], wdwTb,
                     preferred_element_type=jnp.float32)
        hg = a2b[:, g * C_DEL:(g + 1) * C_DEL] * sc2[...] + sh2[...]
        z64 = z64 + hg * Sg[:, :C_DEL]
        z128 = z128 + xgb[:, g * C_IN:(g + 1) * C_IN] * Sg[:, C_DEL:]
    zz = (jnp.dot(z64 + bdw_a[...], Wlin_a[...],
                  preferred_element_type=jnp.float32)
          + jnp.dot(z128 + bdw_b[...], Wlin_b[...],
                    preferred_element_type=jnp.float32)
          + blinr[...])
    zz = jnp.maximum(zz, 0.0)

    oh = (br[...] == jax.lax.broadcasted_iota(jnp.int32, (NB, N_GRAPHS), 1)
          ).astype(jnp.float32)

    @pl.when(pl.program_id(0) == 0)
    def _():
        psum[...] = jnp.zeros_like(psum)
        cnt[...] = jnp.zeros_like(cnt)

    dn = (((0,), (0,)), ((), ()))
    psum[...] += jax.lax.dot_general(oh, zz, dn,
                                     preferred_element_type=jnp.float32)
    cnt[...] += jax.lax.dot_general(oh, jnp.ones((NB, 1), jnp.float32), dn,
                                    preferred_element_type=jnp.float32)
    pooled = psum[...] / jnp.maximum(cnt[...], 1.0)
    out_o[...] = (jnp.dot(pooled, Woutr[...],
                          preferred_element_type=jnp.float32) + boutr[...])


def _p4(a2, w, xg, batch2d, sc2, sh2, sc5, sh5, wdwT,
        bdw_a, bdw_b, Wlin_a, Wlin_b, blinr, Woutr, boutr):
    return pl.pallas_call(
        _p4_body,
        grid=(G_P,),
        in_specs=[
            pl.BlockSpec((NB, K * C_DEL), lambda i: (i, 0)),
            pl.BlockSpec((NB, 256), lambda i: (i, 0)),
            pl.BlockSpec((NB, K * C_IN), lambda i: (i, 0)),
            pl.BlockSpec((NB, 1), lambda i: (i, 0)),
            pl.BlockSpec((1, C_DEL), lambda i: (0, 0)),
            pl.BlockSpec((1, C_DEL), lambda i: (0, 0)),
            pl.BlockSpec((1, 256), lambda i: (0, 0)),
            pl.BlockSpec((1, 256), lambda i: (0, 0)),
            pl.BlockSpec((K, C_TOT), lambda i: (0, 0)),
            pl.BlockSpec((1, C_DEL), lambda i: (0, 0)),
            pl.BlockSpec((1, C_IN), lambda i: (0, 0)),
            pl.BlockSpec((C_DEL, C_OUT), lambda i: (0, 0)),
            pl.BlockSpec((C_IN, C_OUT), lambda i: (0, 0)),
            pl.BlockSpec((1, C_OUT), lambda i: (0, 0)),
            pl.BlockSpec((C_OUT, N_CLS), lambda i: (0, 0)),
            pl.BlockSpec((1, N_CLS), lambda i: (0, 0)),
        ],
        out_specs=pl.BlockSpec((N_GRAPHS, N_CLS), lambda i: (0, 0)),
        out_shape=jax.ShapeDtypeStruct((N_GRAPHS, N_CLS), jnp.float32),
        scratch_shapes=[
            pltpu.VMEM((N_GRAPHS, C_OUT), jnp.float32),
            pltpu.VMEM((N_GRAPHS, 1), jnp.float32),
        ],
    )(a2.reshape(N, K * C_DEL), w, xg.reshape(N, K * C_IN), batch2d,
      sc2, sh2, sc5, sh5, wdwT, bdw_a, bdw_b, Wlin_a, Wlin_b,
      blinr, Woutr, boutr)


def _bn_affine(s, q, n, g, be):
    mean = s / n
    var = q / n - mean * mean
    scale = g.reshape(1, -1) / jnp.sqrt(var + EPS)
    shift = be.reshape(1, -1) - mean * scale
    return scale, shift


def kernel(x, pos, batch, W1, b1, g1, be1, W2, b2, g2, be2, Wl, bl, g3, be3,
           wc1, bc1, g4, be4, wc2, bc2, g5, be5, wdw, bdw, Wlin, blin,
           Wout, bout):
    f32 = jnp.float32

    pos16 = jnp.pad(pos, ((0, 0), (0, 16 - D)))
    posT_pad = jnp.pad(pos, ((0, NPAD - N), (0, 16 - D))).T
    brow = batch.reshape(N, 1)
    bcol = jnp.pad(batch, (0, NPAD - N), constant_values=-1).reshape(1, NPAD)

    idx = _knn(pos16, posT_pad, brow, bcol)
    col = idx.reshape(-1)

    pos128 = jnp.pad(pos, ((0, 0), (0, C_IN - D)))
    pg128 = _sc_gather_pos(pos128, col)
    xg = _sc_gather_x(x, col)
    pg = _p0(pg128)

    W1_16 = jnp.pad(W1, ((0, 16 - D), (0, 0)))
    Wl_exp = jnp.zeros((K, 16, K * K), f32).at[:, :D, :].set(
        Wl.reshape(K, D, K * K)).reshape(256, 256)
    eye16 = jnp.eye(K, dtype=f32)
    T1 = jnp.transpose(wc1, (0, 2, 1))
    Wbig1 = (T1[:, :, None, :] * eye16[:, None, :, None]).reshape(256, 256)
    T2 = jnp.transpose(wc2, (0, 2, 1))
    Wbig2 = (T2[:, :, None, :] * eye16[:, None, :, None]).reshape(256, 256)
    wdwT = jnp.transpose(wdw[:, 0, :])

    pr16 = jnp.repeat(pos16, K, axis=0)
    pr256 = jnp.tile(pos16, (1, K))

    a1, u, s1, q1, su, qu = _p1(pg, pr16, pr256, W1_16,
                                b1.reshape(1, -1), Wl_exp, bl.reshape(1, -1))
    sc1, sh1 = _bn_affine(s1, q1, float(N * K), g1, be1)
    sc3, sh3 = _bn_affine(su, qu, float(N), g3, be3)

    a2, v, s2, q2, sv, qv = _p2(a1, u, sc1, sh1, W2, b2.reshape(1, -1),
                                sc3, sh3, Wbig1, bc1.reshape(1, -1))
    sc2, sh2 = _bn_affine(s2, q2, float(N * K), g2, be2)
    sc4, sh4 = _bn_affine(sv, qv, float(N), g4, be4)

    w, sw, qw = _p3(v, sc4, sh4, Wbig2, bc2.reshape(1, -1))
    sc5, sh5 = _bn_affine(sw, qw, float(N), g5, be5)

    out = _p4(a2, w, xg, batch.reshape(N, 1), sc2, sh2, sc5, sh5, wdwT,
              bdw[:C_DEL].reshape(1, -1), bdw[C_DEL:].reshape(1, -1),
              Wlin[:C_DEL], Wlin[C_DEL:], blin.reshape(1, -1),
              Wout, bout.reshape(1, -1))
    return out


# speedup vs baseline: 9.6346x; 3.6652x over previous
import functools

import jax
import jax.numpy as jnp
from jax.experimental import pallas as pl
from jax.experimental.pallas import tpu as pltpu
from jax.experimental.pallas import tpu_sc as plsc

N = 10000
D = 3
K = 16
C_IN = 128
C_DEL = 64
C_TOT = C_IN + C_DEL
C_OUT = 128
N_CLS = 40
N_GRAPHS = 16
EPS = 1e-5
NPAD = 10240

R_KNN = 200
NB = 400
RB = NB * K
G_KNN = N // R_KNN
G_P = N // NB

CH = 128
NW = 32
NCHUNK = (N * K) // CH
SC_ITERS = (NCHUNK + NW - 1) // NW


def _elu(x):
    return jnp.where(x > 0, x, jnp.exp(x) - 1.0)


WWIN = 2048
BIG = 3e38


def _topk_cols(d2, width, base):
    iota = jax.lax.broadcasted_iota(jnp.int32, d2.shape, 1)
    cols = []
    for _ in range(K):
        m = jnp.min(d2, axis=1, keepdims=True)
        sel = jnp.where(d2 == m, iota, jnp.int32(width))
        ig = jnp.min(sel, axis=1, keepdims=True)
        cols.append(ig + base)
        d2 = jnp.where(iota == ig, jnp.float32(jnp.inf), d2)
    return jnp.concatenate(cols, axis=1)


def _knn_full_body(prow, pcolT, brow, bcol, idx_out):
    p = prow[...]
    pt = pcolT[...]
    sqr = jnp.sum(p * p, axis=1, keepdims=True)
    sqc = jnp.sum(pt * pt, axis=0, keepdims=True)
    dot = jnp.dot(p, pt, preferred_element_type=jnp.float32)
    d2 = sqr + sqc - 2.0 * dot
    d2 = jnp.where(brow[...] == bcol[...], d2, BIG)
    idx_out[...] = _topk_cols(d2, NPAD, jnp.int32(0))


def _knn_win_body(offs, prow, pcolT, brow, bcol, idx_out):
    off = pl.multiple_of(offs[pl.program_id(0)], 128)
    p = prow[...]
    pt = pcolT[:, pl.ds(off, WWIN)]
    bc = bcol[:, pl.ds(off, WWIN)]
    sqr = jnp.sum(p * p, axis=1, keepdims=True)
    sqc = jnp.sum(pt * pt, axis=0, keepdims=True)
    dot = jnp.dot(p, pt, preferred_element_type=jnp.float32)
    d2 = sqr + sqc - 2.0 * dot
    d2 = jnp.where(brow[...] == bc, d2, BIG)
    idx_out[...] = _topk_cols(d2, WWIN, off)


def _knn(pos16, posT_pad, brow, bcol, offs, ok):
    common = dict(
        out_specs=pl.BlockSpec((R_KNN, K), lambda i: (i, 0)),
        out_shape=jax.ShapeDtypeStruct((N, K), jnp.int32),
        grid=(G_KNN,),
    )
    full = pl.pallas_call(
        _knn_full_body,
        in_specs=[
            pl.BlockSpec((R_KNN, 16), lambda i: (i, 0)),
            pl.BlockSpec((16, NPAD), lambda i: (0, 0)),
            pl.BlockSpec((R_KNN, 1), lambda i: (i, 0)),
            pl.BlockSpec((1, NPAD), lambda i: (0, 0)),
        ],
        **common,
    )
    win = pl.pallas_call(
        _knn_win_body,
        in_specs=[
            pl.BlockSpec(memory_space=pltpu.SMEM),
            pl.BlockSpec((R_KNN, 16), lambda i: (i, 0)),
            pl.BlockSpec((16, NPAD), lambda i: (0, 0)),
            pl.BlockSpec((R_KNN, 1), lambda i: (i, 0)),
            pl.BlockSpec((1, NPAD), lambda i: (0, 0)),
        ],
        **common,
    )
    return jax.lax.cond(
        ok,
        lambda: win(offs, pos16, posT_pad, brow, bcol),
        lambda: full(pos16, posT_pad, brow, bcol),
    )


def _make_sc_gather(feat):
    mesh = plsc.VectorSubcoreMesh(core_axis_name="c", subcore_axis_name="s")

    @functools.partial(
        pl.kernel,
        out_type=jax.ShapeDtypeStruct((N * K, feat), jnp.float32),
        mesh=mesh,
        scratch_types=[
            pltpu.VMEM((CH,), jnp.int32),
            pltpu.VMEM((CH, feat), jnp.float32),
            pltpu.SemaphoreType.DMA,
        ],
    )
    def gather(table_hbm, col_hbm, out_hbm, idx_v, rows_v, sem):
        wid = jax.lax.axis_index("s") * 2 + jax.lax.axis_index("c")

        def body(j, carry):
            c = j * NW + wid

            @pl.when(c < NCHUNK)
            def _():
                base = c * CH
                pltpu.sync_copy(col_hbm.at[pl.ds(base, CH)], idx_v)
                pltpu.async_copy(table_hbm.at[idx_v], rows_v, sem).wait()
                pltpu.sync_copy(rows_v, out_hbm.at[pl.ds(base, CH)])

            return carry

        jax.lax.fori_loop(0, SC_ITERS, body, 0)

    return gather


def _sc_gather_pos(table, col):
    return _make_sc_gather(C_IN)(table, col)


def _sc_gather_x(table, col):
    return _make_sc_gather(C_IN)(table, col)


def _p0_body(pg128, pg_o):
    pg_o[...] = pg128[...][:, :16]


def _p0(pg128):
    return pl.pallas_call(
        _p0_body,
        grid=(G_P,),
        in_specs=[pl.BlockSpec((RB, C_IN), lambda i: (i, 0))],
        out_specs=pl.BlockSpec((RB, 16), lambda i: (i, 0)),
        out_shape=jax.ShapeDtypeStruct((N * K, 16), jnp.float32),
    )(pg128)


def _p1_body(pg16, pg256, pr16, pr256, W1r, b1r, Wlr, blr,
             a1_o, u_o, s1_o, q1_o, su_o, qu_o):
    pd16 = pg16[...] - pr16[...]
    a1 = _elu(jnp.dot(pd16, W1r[...], preferred_element_type=jnp.float32)
              + b1r[...])
    a1_o[...] = a1
    pd256 = pg256[...] - pr256[...]
    u = _elu(jnp.dot(pd256, Wlr[...], preferred_element_type=jnp.float32)
             + blr[...])
    u_o[...] = u

    @pl.when(pl.program_id(0) == 0)
    def _():
        s1_o[...] = jnp.zeros_like(s1_o)
        q1_o[...] = jnp.zeros_like(q1_o)
        su_o[...] = jnp.zeros_like(su_o)
        qu_o[...] = jnp.zeros_like(qu_o)

    s1_o[...] += jnp.sum(a1, axis=0, keepdims=True)
    q1_o[...] += jnp.sum(a1 * a1, axis=0, keepdims=True)
    su_o[...] += jnp.sum(u, axis=0, keepdims=True)
    qu_o[...] += jnp.sum(u * u, axis=0, keepdims=True)


def _p1(pg, pr16, pr256, W1_16, b1r, Wl_exp, blr):
    return pl.pallas_call(
        _p1_body,
        grid=(G_P,),
        in_specs=[
            pl.BlockSpec((RB, 16), lambda i: (i, 0)),
            pl.BlockSpec((NB, 256), lambda i: (i, 0)),
            pl.BlockSpec((RB, 16), lambda i: (i, 0)),
            pl.BlockSpec((NB, 256), lambda i: (i, 0)),
            pl.BlockSpec((16, C_DEL), lambda i: (0, 0)),
            pl.BlockSpec((1, C_DEL), lambda i: (0, 0)),
            pl.BlockSpec((256, 256), lambda i: (0, 0)),
            pl.BlockSpec((1, 256), lambda i: (0, 0)),
        ],
        out_specs=[
            pl.BlockSpec((RB, C_DEL), lambda i: (i, 0)),
            pl.BlockSpec((NB, 256), lambda i: (i, 0)),
            pl.BlockSpec((1, C_DEL), lambda i: (0, 0)),
            pl.BlockSpec((1, C_DEL), lambda i: (0, 0)),
            pl.BlockSpec((1, 256), lambda i: (0, 0)),
            pl.BlockSpec((1, 256), lambda i: (0, 0)),
        ],
        out_shape=[
            jax.ShapeDtypeStruct((N * K, C_DEL), jnp.float32),
            jax.ShapeDtypeStruct((N, 256), jnp.float32),
            jax.ShapeDtypeStruct((1, C_DEL), jnp.float32),
            jax.ShapeDtypeStruct((1, C_DEL), jnp.float32),
            jax.ShapeDtypeStruct((1, 256), jnp.float32),
            jax.ShapeDtypeStruct((1, 256), jnp.float32),
        ],
    )(pg, pg.reshape(N, 256), pr16, pr256, W1_16, b1r, Wl_exp, blr)


def _p2_body(a1r, ur, sc1, sh1, W2r, b2r, sc3, sh3, Wb1, bc1r,
             a2_o, v_o, s2_o, q2_o, sv_o, qv_o):
    h1 = a1r[...] * sc1[...] + sh1[...]
    a2 = _elu(jnp.dot(h1, W2r[...], preferred_element_type=jnp.float32)
              + b2r[...])
    a2_o[...] = a2
    ub = ur[...] * sc3[...] + sh3[...]
    v = _elu(jnp.dot(ub, Wb1[...], preferred_element_type=jnp.float32)
             + bc1r[...])
    v_o[...] = v

    @pl.when(pl.program_id(0) == 0)
    def _():
        s2_o[...] = jnp.zeros_like(s2_o)
        q2_o[...] = jnp.zeros_like(q2_o)
        sv_o[...] = jnp.zeros_like(sv_o)
        qv_o[...] = jnp.zeros_like(qv_o)

    s2_o[...] += jnp.sum(a2, axis=0, keepdims=True)
    q2_o[...] += jnp.sum(a2 * a2, axis=0, keepdims=True)
    sv_o[...] += jnp.sum(v, axis=0, keepdims=True)
    qv_o[...] += jnp.sum(v * v, axis=0, keepdims=True)


def _p2(a1, u, sc1, sh1, W2, b2r, sc3, sh3, Wbig1, bc1r):
    return pl.pallas_call(
        _p2_body,
        grid=(G_P,),
        in_specs=[
            pl.BlockSpec((RB, C_DEL), lambda i: (i, 0)),
            pl.BlockSpec((NB, 256), lambda i: (i, 0)),
            pl.BlockSpec((1, C_DEL), lambda i: (0, 0)),
            pl.BlockSpec((1, C_DEL), lambda i: (0, 0)),
            pl.BlockSpec((C_DEL, C_DEL), lambda i: (0, 0)),
            pl.BlockSpec((1, C_DEL), lambda i: (0, 0)),
            pl.BlockSpec((1, 256), lambda i: (0, 0)),
            pl.BlockSpec((1, 256), lambda i: (0, 0)),
            pl.BlockSpec((256, 256), lambda i: (0, 0)),
            pl.BlockSpec((1, 256), lambda i: (0, 0)),
        ],
        out_specs=[
            pl.BlockSpec((RB, C_DEL), lambda i: (i, 0)),
            pl.BlockSpec((NB, 256), lambda i: (i, 0)),
            pl.BlockSpec((1, C_DEL), lambda i: (0, 0)),
            pl.BlockSpec((1, C_DEL), lambda i: (0, 0)),
            pl.BlockSpec((1, 256), lambda i: (0, 0)),
            pl.BlockSpec((1, 256), lambda i: (0, 0)),
        ],
        out_shape=[
            jax.ShapeDtypeStruct((N * K, C_DEL), jnp.float32),
            jax.ShapeDtypeStruct((N, 256), jnp.float32),
            jax.ShapeDtypeStruct((1, C_DEL), jnp.float32),
            jax.ShapeDtypeStruct((1, C_DEL), jnp.float32),
            jax.ShapeDtypeStruct((1, 256), jnp.float32),
            jax.ShapeDtypeStruct((1, 256), jnp.float32),
        ],
    )(a1, u, sc1, sh1, W2, b2r, sc3, sh3, Wbig1, bc1r)


def _p3_body(vr, sc4, sh4, Wb2, bc2r, w_o, sw_o, qw_o):
    vb = vr[...] * sc4[...] + sh4[...]
    w = jnp.dot(vb, Wb2[...], preferred_element_type=jnp.float32) + bc2r[...]
    w_o[...] = w

    @pl.when(pl.program_id(0) == 0)
    def _():
        sw_o[...] = jnp.zeros_like(sw_o)
        qw_o[...] = jnp.zeros_like(qw_o)

    sw_o[...] += jnp.sum(w, axis=0, keepdims=True)
    qw_o[...] += jnp.sum(w * w, axis=0, keepdims=True)


def _p3(v, sc4, sh4, Wbig2, bc2r):
    return pl.pallas_call(
        _p3_body,
        grid=(G_P,),
        in_specs=[
            pl.BlockSpec((NB, 256), lambda i: (i, 0)),
            pl.BlockSpec((1, 256), lambda i: (0, 0)),
            pl.BlockSpec((1, 256), lambda i: (0, 0)),
            pl.BlockSpec((256, 256), lambda i: (0, 0)),
            pl.BlockSpec((1, 256), lambda i: (0, 0)),
        ],
        out_specs=[
            pl.BlockSpec((NB, 256), lambda i: (i, 0)),
            pl.BlockSpec((1, 256), lambda i: (0, 0)),
            pl.BlockSpec((1, 256), lambda i: (0, 0)),
        ],
        out_shape=[
            jax.ShapeDtypeStruct((N, 256), jnp.float32),
            jax.ShapeDtypeStruct((1, 256), jnp.float32),
            jax.ShapeDtypeStruct((1, 256), jnp.float32),
        ],
    )(v, sc4, sh4, Wbig2, bc2r)


def _p4_body(a2v, wr, xgv, br, sc2, sh2, sc5, sh5, wdwT,
             bdw_a, bdw_b, Wlin_a, Wlin_b, blinr, Woutr, boutr,
             out_o, psum, cnt):
    Tm = wr[...] * sc5[...] + sh5[...]
    a2b = a2v[...]
    xgb = xgv[...]
    wdwTb = wdwT[...]
    z64 = jnp.zeros((NB, C_DEL), jnp.float32)
    z128 = jnp.zeros((NB, C_IN), jnp.float32)
    for g in range(K):
        Sg = jnp.dot(Tm[:, g * 16:(g + 1) * 16], wdwTb,
                     preferred_element_type=jnp.float32)
        hg = a2b[:, g * C_DEL:(g + 1) * C_DEL] * sc2[...] + sh2[...]
        z64 = z64 + hg * Sg[:, :C_DEL]
        z128 = z128 + xgb[:, g * C_IN:(g + 1) * C_IN] * Sg[:, C_DEL:]
    zz = (jnp.dot(z64 + bdw_a[...], Wlin_a[...],
                  preferred_element_type=jnp.float32)
          + jnp.dot(z128 + bdw_b[...], Wlin_b[...],
                    preferred_element_type=jnp.float32)
          + blinr[...])
    zz = jnp.maximum(zz, 0.0)

    oh = (br[...] == jax.lax.broadcasted_iota(jnp.int32, (NB, N_GRAPHS), 1)
          ).astype(jnp.float32)

    @pl.when(pl.program_id(0) == 0)
    def _():
        psum[...] = jnp.zeros_like(psum)
        cnt[...] = jnp.zeros_like(cnt)

    dn = (((0,), (0,)), ((), ()))
    psum[...] += jax.lax.dot_general(oh, zz, dn,
                                     preferred_element_type=jnp.float32)
    cnt[...] += jax.lax.dot_general(oh, jnp.ones((NB, 1), jnp.float32), dn,
                                    preferred_element_type=jnp.float32)
    pooled = psum[...] / jnp.maximum(cnt[...], 1.0)
    out_o[...] = (jnp.dot(pooled, Woutr[...],
                          preferred_element_type=jnp.float32) + boutr[...])


def _p4(a2, w, xg, batch2d, sc2, sh2, sc5, sh5, wdwT,
        bdw_a, bdw_b, Wlin_a, Wlin_b, blinr, Woutr, boutr):
    return pl.pallas_call(
        _p4_body,
        grid=(G_P,),
        in_specs=[
            pl.BlockSpec((NB, K * C_DEL), lambda i: (i, 0)),
            pl.BlockSpec((NB, 256), lambda i: (i, 0)),
            pl.BlockSpec((NB, K * C_IN), lambda i: (i, 0)),
            pl.BlockSpec((NB, 1), lambda i: (i, 0)),
            pl.BlockSpec((1, C_DEL), lambda i: (0, 0)),
            pl.BlockSpec((1, C_DEL), lambda i: (0, 0)),
            pl.BlockSpec((1, 256), lambda i: (0, 0)),
            pl.BlockSpec((1, 256), lambda i: (0, 0)),
            pl.BlockSpec((K, C_TOT), lambda i: (0, 0)),
            pl.BlockSpec((1, C_DEL), lambda i: (0, 0)),
            pl.BlockSpec((1, C_IN), lambda i: (0, 0)),
            pl.BlockSpec((C_DEL, C_OUT), lambda i: (0, 0)),
            pl.BlockSpec((C_IN, C_OUT), lambda i: (0, 0)),
            pl.BlockSpec((1, C_OUT), lambda i: (0, 0)),
            pl.BlockSpec((C_OUT, N_CLS), lambda i: (0, 0)),
            pl.BlockSpec((1, N_CLS), lambda i: (0, 0)),
        ],
        out_specs=pl.BlockSpec((N_GRAPHS, N_CLS), lambda i: (0, 0)),
        out_shape=jax.ShapeDtypeStruct((N_GRAPHS, N_CLS), jnp.float32),
        scratch_shapes=[
            pltpu.VMEM((N_GRAPHS, C_OUT), jnp.float32),
            pltpu.VMEM((N_GRAPHS, 1), jnp.float32),
        ],
    )(a2.reshape(N, K * C_DEL), w, xg.reshape(N, K * C_IN), batch2d,
      sc2, sh2, sc5, sh5, wdwT, bdw_a, bdw_b, Wlin_a, Wlin_b,
      blinr, Woutr, boutr)


def _bn_affine(s, q, n, g, be):
    mean = s / n
    var = q / n - mean * mean
    scale = g.reshape(1, -1) / jnp.sqrt(var + EPS)
    shift = be.reshape(1, -1) - mean * scale
    return scale, shift


def kernel(x, pos, batch, W1, b1, g1, be1, W2, b2, g2, be2, Wl, bl, g3, be3,
           wc1, bc1, g4, be4, wc2, bc2, g5, be5, wdw, bdw, Wlin, blin,
           Wout, bout):
    f32 = jnp.float32

    pos16 = jnp.pad(pos, ((0, 0), (0, 16 - D)))
    posT_pad = jnp.pad(pos, ((0, NPAD - N), (0, 16 - D))).T
    brow = batch.reshape(N, 1)
    bcol = jnp.pad(batch, (0, NPAD - N), constant_values=-1).reshape(1, NPAD)

    gr = jnp.arange(N_GRAPHS)
    seg_start = jnp.searchsorted(batch, gr, side="left")
    seg_end = jnp.searchsorted(batch, gr, side="right")
    sizes = seg_end - seg_start
    bfirst = batch[0::R_KNN]
    blast = batch[R_KNN - 1::R_KNN]
    ws = seg_start[bfirst]
    we = seg_end[blast]
    offs = jnp.minimum((ws // 128) * 128, NPAD - WWIN).astype(jnp.int32)
    ok = jnp.all(we <= offs + WWIN) & (jnp.min(sizes) >= K)

    idx = _knn(pos16, posT_pad, brow, bcol, offs, ok)
    col = idx.reshape(-1)

    pos128 = jnp.pad(pos, ((0, 0), (0, C_IN - D)))
    pg128 = _sc_gather_pos(pos128, col)
    xg = _sc_gather_x(x, col)
    pg = _p0(pg128)

    W1_16 = jnp.pad(W1, ((0, 16 - D), (0, 0)))
    Wl_exp = jnp.zeros((K, 16, K * K), f32).at[:, :D, :].set(
        Wl.reshape(K, D, K * K)).reshape(256, 256)
    eye16 = jnp.eye(K, dtype=f32)
    T1 = jnp.transpose(wc1, (0, 2, 1))
    Wbig1 = (T1[:, :, None, :] * eye16[:, None, :, None]).reshape(256, 256)
    T2 = jnp.transpose(wc2, (0, 2, 1))
    Wbig2 = (T2[:, :, None, :] * eye16[:, None, :, None]).reshape(256, 256)
    wdwT = jnp.transpose(wdw[:, 0, :])

    pr16 = jnp.repeat(pos16, K, axis=0)
    pr256 = jnp.tile(pos16, (1, K))

    a1, u, s1, q1, su, qu = _p1(pg, pr16, pr256, W1_16,
                                b1.reshape(1, -1), Wl_exp, bl.reshape(1, -1))
    sc1, sh1 = _bn_affine(s1, q1, float(N * K), g1, be1)
    sc3, sh3 = _bn_affine(su, qu, float(N), g3, be3)

    a2, v, s2, q2, sv, qv = _p2(a1, u, sc1, sh1, W2, b2.reshape(1, -1),
                                sc3, sh3, Wbig1, bc1.reshape(1, -1))
    sc2, sh2 = _bn_affine(s2, q2, float(N * K), g2, be2)
    sc4, sh4 = _bn_affine(sv, qv, float(N), g4, be4)

    w, sw, qw = _p3(v, sc4, sh4, Wbig2, bc2.reshape(1, -1))
    sc5, sh5 = _bn_affine(sw, qw, float(N), g5, be5)

    out = _p4(a2, w, xg, batch.reshape(N, 1), sc2, sh2, sc5, sh5, wdwT,
              bdw[:C_DEL].reshape(1, -1), bdw[C_DEL:].reshape(1, -1),
              Wlin[:C_DEL], Wlin[C_DEL:], blin.reshape(1, -1),
              Wout, bout.reshape(1, -1))
    return out
